# baseline (speedup 1.0000x reference)
import functools

import jax
import jax.numpy as jnp
import numpy as np
from jax import lax
from jax.experimental import pallas as pl
from jax.experimental.pallas import tpu as pltpu
import jax.experimental.pallas.tpu_sc as plsc

NUM_SPH = 7
NUM_RAD = 6
CUTOFF = 5.0
P_ENV = 6
A_ENV = -(P_ENV + 1) * (P_ENV + 2) / 2.0
B_ENV = float(P_ENV * (P_ENV + 2))
C_ENV = -P_ENV * (P_ENV + 1) / 2.0

NCOL = NUM_SPH * NUM_RAD
NPAD = 48


def _jn_np(x, l):
    x = np.asarray(x, dtype=np.float64)
    j0 = np.sin(x) / x
    if l == 0:
        return j0
    j1 = np.sin(x) / x ** 2 - np.cos(x) / x
    if l == 1:
        return j1
    jm1, j = j0, j1
    for ll in range(1, l):
        jm1, j = j, (2 * ll + 1) / x * j - jm1
    return j


def _jn_zeros(n, k):
    zeros = np.zeros((n, k))
    for l in range(n):
        grid = np.linspace(0.5 + 0.5 * l, 45.0 + 5.0 * l, 200001)
        vals = _jn_np(grid, l)
        sign_change = np.where(np.sign(vals[:-1]) * np.sign(vals[1:]) < 0)[0][:k]
        for i, ii in enumerate(sign_change):
            lo, hi = grid[ii], grid[ii + 1]
            flo = _jn_np(lo, l)
            for _ in range(60):
                mid = 0.5 * (lo + hi)
                fmid = _jn_np(mid, l)
                if flo * fmid <= 0:
                    hi = mid
                else:
                    lo, flo = mid, fmid
            zeros[l, i] = 0.5 * (lo + hi)
    return zeros


_Z = _jn_zeros(NUM_SPH, NUM_RAD)
_NORM = np.stack([1.0 / np.sqrt(0.5 * _jn_np(_Z[l], l + 1) ** 2) for l in range(NUM_SPH)])
_SPH_PREF = np.array([np.sqrt((2 * l + 1) / (4 * np.pi)) for l in range(NUM_SPH)])

_ZPAD = np.ones((NPAD,), np.float32)
_ZPAD[:NCOL] = _Z.reshape(-1)
_NORMPAD = np.zeros((NPAD,), np.float32)
_NORMPAD[:NCOL] = _NORM.reshape(-1)
for _c in range(NCOL):
    _NORMPAD[_c] *= np.float32(_SPH_PREF[_c // NUM_RAD])

_PLC = np.zeros((NUM_SPH, NUM_SPH))
_PLC[0, 0] = 1.0
_PLC[1, 1] = 1.0
for _l in range(1, NUM_SPH - 1):
    _PLC[_l + 1, 1:] = (2 * _l + 1) * _PLC[_l, :6]
    _PLC[_l + 1] -= _l * _PLC[_l - 1]
    _PLC[_l + 1] /= _l + 1
_PLPAD = np.zeros((NPAD, 8), np.float32)
for _c in range(NCOL):
    _PLPAD[_c, :NUM_SPH] = _PLC[_c // NUM_RAD].astype(np.float32)


def _make_sc_gather(E, T):
    NC = 2
    NW = 32
    ROWS_W = T // NW
    SUB = 120
    CH = 3000
    NSUB = CH // SUB
    NCHUNK = ROWS_W // CH
    assert T == NW * NCHUNK * CH and CH == NSUB * SUB

    mesh = plsc.VectorSubcoreMesh(core_axis_name="c", subcore_axis_name="s")

    @functools.partial(
        pl.kernel,
        out_type=jax.ShapeDtypeStruct((T,), jnp.float32),
        mesh=mesh,
        scratch_types=[
            pltpu.VMEM((CH,), jnp.int32),
            pltpu.VMEM((CH,), jnp.float32),
            pltpu.SemaphoreType.DMA,
        ],
    )
    def sc_gather(dist_hbm, idx_hbm, out_hbm, idx_v, val_v, sem):
        wid = lax.axis_index("s") * NC + lax.axis_index("c")
        row0 = wid * ROWS_W

        def chunk_body(i, carry):
            base = row0 + i * CH
            pltpu.sync_copy(idx_hbm.at[pl.ds(base, CH)], idx_v)
            cps = [
                pltpu.async_copy(
                    dist_hbm.at[idx_v.at[pl.ds(j * SUB, SUB)]],
                    val_v.at[pl.ds(j * SUB, SUB)],
                    sem,
                )
                for j in range(NSUB)
            ]
            for cp in cps:
                cp.wait()
            pltpu.sync_copy(val_v, out_hbm.at[pl.ds(base, CH)])
            return carry

        lax.fori_loop(0, NCHUNK, chunk_body, 0)

    return sc_gather


_PIO2_HI = np.float32(1.57079625129699707031)
_PIO2_LO = np.float32(7.54978941586159635335e-08)
_S1, _S2, _S3 = np.float32(-1.6666654611e-1), np.float32(8.3321608736e-3), np.float32(-1.9515295891e-4)
_C1, _C2, _C3 = np.float32(-0.5), np.float32(4.166664568298827e-2), np.float32(-1.388731625493765e-3)
_C4 = np.float32(2.443315711809948e-5)


def _sincos(y):
    q = y * np.float32(2.0 / np.pi)
    nf = jnp.floor(q + 0.5)
    ni = nf.astype(jnp.int32)
    r = (y - nf * _PIO2_HI) - nf * _PIO2_LO
    r2 = r * r
    sp = r + r * r2 * (_S1 + r2 * (_S2 + r2 * _S3))
    cp = 1.0 + r2 * (_C1 + r2 * (_C2 + r2 * (_C3 + r2 * _C4)))
    odd = (ni & 1) == 1
    sin_sel = jnp.where(odd, cp, sp)
    cos_sel = jnp.where(odd, sp, cp)
    sbit = jax.lax.shift_left(ni & 2, 30)
    cbit = jax.lax.shift_left((ni + 1) & 2, 30)
    sin = lax.bitcast_convert_type(
        lax.bitcast_convert_type(sin_sel, jnp.int32) ^ sbit, jnp.float32
    )
    cos = lax.bitcast_convert_type(
        lax.bitcast_convert_type(cos_sel, jnp.int32) ^ cbit, jnp.float32
    )
    return sin, cos


def _basis_body(z_ref, n_ref, p_ref, d_ref, a_ref, o_ref):
    for k in range(d_ref.shape[0] // 128):
        sl = pl.ds(k * 128, 128)
        _basis_chunk(z_ref, n_ref, p_ref, d_ref[sl], a_ref[sl], o_ref, sl)


def _basis_chunk(z_ref, n_ref, p_ref, d, a, o_ref, sl):
    x = d * (1.0 / CUTOFF)
    x2 = x * x
    x4 = x2 * x2
    x5 = x4 * x
    env = 1.0 / x + A_ENV * x5 + B_ENV * x5 * x + C_ENV * x5 * x2
    y = z_ref[:] * x[None, :]
    inv_y = 1.0 / y
    sy, cy = _sincos(y)
    j0 = sy * inv_y
    j1 = sy * inv_y * inv_y - cy * inv_y
    lcol = lax.broadcasted_iota(jnp.int32, y.shape, 0) // NUM_RAD
    res = jnp.where(lcol == 0, j0, j1)
    jm1, j = j0, j1
    for s in range(1, NUM_SPH - 1):
        jm1, j = j, (2 * s + 1) * inv_y * j - jm1
        res = jnp.where(lcol == s + 1, j, res)

    zc = _sincos(a)[1]
    z2 = zc * zc
    z3 = z2 * zc
    z4 = z2 * z2
    z5 = z3 * z2
    z6 = z3 * z3
    zp = jnp.concatenate(
        [
            jnp.ones_like(zc)[None, :], zc[None, :], z2[None, :], z3[None, :],
            z4[None, :], z5[None, :], z6[None, :], jnp.zeros_like(zc)[None, :],
        ],
        axis=0,
    )
    cb = jax.lax.dot(
        p_ref[:, :], zp,
        precision=jax.lax.Precision.HIGHEST,
        preferred_element_type=jnp.float32,
    )

    full = res * n_ref[:] * env[None, :] * cb
    o_ref[:, sl] = full[:NCOL, :]


def kernel(dist, angle, idx_kj):
    E = dist.shape[0]
    T = angle.shape[0]

    sc_gather = _make_sc_gather(E, T)
    dist_t = sc_gather(dist, idx_kj)

    BT = 2048
    out = pl.pallas_call(
        _basis_body,
        grid=(pl.cdiv(T, BT),),
        in_specs=[
            pl.BlockSpec((NPAD, 1), lambda i: (0, 0)),
            pl.BlockSpec((NPAD, 1), lambda i: (0, 0)),
            pl.BlockSpec((NPAD, 8), lambda i: (0, 0)),
            pl.BlockSpec((BT,), lambda i: (i,)),
            pl.BlockSpec((BT,), lambda i: (i,)),
        ],
        out_specs=pl.BlockSpec((NCOL, BT), lambda i: (0, i)),
        out_shape=jax.ShapeDtypeStruct((NCOL, T), jnp.float32),
    )(
        jnp.asarray(_ZPAD)[:, None],
        jnp.asarray(_NORMPAD)[:, None],
        jnp.asarray(_PLPAD),
        dist_t,
        angle,
    )
    return out.T


# speedup vs baseline: 1.0808x; 1.0808x over previous
import functools

import jax
import jax.numpy as jnp
import numpy as np
from jax import lax
from jax.experimental import pallas as pl
from jax.experimental.pallas import tpu as pltpu
import jax.experimental.pallas.tpu_sc as plsc

NUM_SPH = 7
NUM_RAD = 6
CUTOFF = 5.0
P_ENV = 6
A_ENV = -(P_ENV + 1) * (P_ENV + 2) / 2.0
B_ENV = float(P_ENV * (P_ENV + 2))
C_ENV = -P_ENV * (P_ENV + 1) / 2.0

NCOL = NUM_SPH * NUM_RAD
NPAD = 48


def _jn_np(x, l):
    x = np.asarray(x, dtype=np.float64)
    j0 = np.sin(x) / x
    if l == 0:
        return j0
    j1 = np.sin(x) / x ** 2 - np.cos(x) / x
    if l == 1:
        return j1
    jm1, j = j0, j1
    for ll in range(1, l):
        jm1, j = j, (2 * ll + 1) / x * j - jm1
    return j


def _jn_zeros(n, k):
    zeros = np.zeros((n, k))
    for l in range(n):
        grid = np.linspace(0.5 + 0.5 * l, 45.0 + 5.0 * l, 200001)
        vals = _jn_np(grid, l)
        sign_change = np.where(np.sign(vals[:-1]) * np.sign(vals[1:]) < 0)[0][:k]
        for i, ii in enumerate(sign_change):
            lo, hi = grid[ii], grid[ii + 1]
            flo = _jn_np(lo, l)
            for _ in range(60):
                mid = 0.5 * (lo + hi)
                fmid = _jn_np(mid, l)
                if flo * fmid <= 0:
                    hi = mid
                else:
                    lo, flo = mid, fmid
            zeros[l, i] = 0.5 * (lo + hi)
    return zeros


_Z = _jn_zeros(NUM_SPH, NUM_RAD)
_NORM = np.stack([1.0 / np.sqrt(0.5 * _jn_np(_Z[l], l + 1) ** 2) for l in range(NUM_SPH)])
_SPH_PREF = np.array([np.sqrt((2 * l + 1) / (4 * np.pi)) for l in range(NUM_SPH)])

_ZPAD = np.ones((NPAD,), np.float32)
_ZPAD[:NCOL] = _Z.reshape(-1)
_NORMPAD = np.zeros((NPAD,), np.float32)
_NORMPAD[:NCOL] = _NORM.reshape(-1)
for _c in range(NCOL):
    _NORMPAD[_c] *= np.float32(_SPH_PREF[_c // NUM_RAD])


def _make_sc_gather(E, T):
    NC = 2
    NW = 32
    ROWS_W = T // NW
    SUB = 120
    CH = 3000
    NSUB = CH // SUB
    NCHUNK = ROWS_W // CH
    assert T == NW * NCHUNK * CH and CH == NSUB * SUB

    mesh = plsc.VectorSubcoreMesh(core_axis_name="c", subcore_axis_name="s")

    @functools.partial(
        pl.kernel,
        out_type=jax.ShapeDtypeStruct((T,), jnp.float32),
        mesh=mesh,
        scratch_types=[
            pltpu.VMEM((CH,), jnp.int32),
            pltpu.VMEM((CH,), jnp.float32),
            pltpu.SemaphoreType.DMA,
        ],
    )
    def sc_gather(dist_hbm, idx_hbm, out_hbm, idx_v, val_v, sem):
        wid = lax.axis_index("s") * NC + lax.axis_index("c")
        row0 = wid * ROWS_W

        def chunk_body(i, carry):
            base = row0 + i * CH
            pltpu.sync_copy(idx_hbm.at[pl.ds(base, CH)], idx_v)
            cps = [
                pltpu.async_copy(
                    dist_hbm.at[idx_v.at[pl.ds(j * SUB, SUB)]],
                    val_v.at[pl.ds(j * SUB, SUB)],
                    sem,
                )
                for j in range(NSUB)
            ]
            for cp in cps:
                cp.wait()
            pltpu.sync_copy(val_v, out_hbm.at[pl.ds(base, CH)])
            return carry

        lax.fori_loop(0, NCHUNK, chunk_body, 0)

    return sc_gather


_PIO2_HI = np.float32(1.57079625129699707031)
_PIO2_LO = np.float32(7.54978941586159635335e-08)
_S1, _S2, _S3 = np.float32(-1.6666654611e-1), np.float32(8.3321608736e-3), np.float32(-1.9515295891e-4)
_C1, _C2, _C3 = np.float32(-0.5), np.float32(4.166664568298827e-2), np.float32(-1.388731625493765e-3)
_C4 = np.float32(2.443315711809948e-5)


def _sincos(y):
    q = y * np.float32(2.0 / np.pi)
    nf = jnp.floor(q + 0.5)
    ni = nf.astype(jnp.int32)
    r = (y - nf * _PIO2_HI) - nf * _PIO2_LO
    r2 = r * r
    sp = r + r * r2 * (_S1 + r2 * (_S2 + r2 * _S3))
    cp = 1.0 + r2 * (_C1 + r2 * (_C2 + r2 * (_C3 + r2 * _C4)))
    odd = (ni & 1) == 1
    sin_sel = jnp.where(odd, cp, sp)
    cos_sel = jnp.where(odd, sp, cp)
    sbit = jax.lax.shift_left(ni & 2, 30)
    cbit = jax.lax.shift_left((ni + 1) & 2, 30)
    sin = lax.bitcast_convert_type(
        lax.bitcast_convert_type(sin_sel, jnp.int32) ^ sbit, jnp.float32
    )
    cos = lax.bitcast_convert_type(
        lax.bitcast_convert_type(cos_sel, jnp.int32) ^ cbit, jnp.float32
    )
    return sin, cos


def _basis_body(z_ref, n_ref, d_ref, a_ref, o_ref):
    for k in range(d_ref.shape[0] // 128):
        sl = pl.ds(k * 128, 128)
        _basis_chunk(z_ref, n_ref, d_ref[sl], a_ref[sl], o_ref, sl)


def _basis_chunk(z_ref, n_ref, d, a, o_ref, sl):
    x = d * (1.0 / CUTOFF)
    x2 = x * x
    x4 = x2 * x2
    x5 = x4 * x
    env = 1.0 / x + A_ENV * x5 + B_ENV * x5 * x + C_ENV * x5 * x2
    y = z_ref[:] * x[None, :]
    inv_y = 1.0 / y
    sy, cy = _sincos(y)
    j0 = sy * inv_y
    j1 = sy * inv_y * inv_y - cy * inv_y
    lcol = lax.broadcasted_iota(jnp.int32, y.shape, 0) // NUM_RAD
    res = jnp.where(lcol == 0, j0, j1)
    jm1, j = j0, j1
    for s in range(1, NUM_SPH - 1):
        jm1, j = j, (2 * s + 1) * inv_y * j - jm1
        res = jnp.where(lcol == s + 1, j, res)

    zc = _sincos(a)[1]
    ps = [jnp.ones_like(zc), zc]
    for l in range(1, NUM_SPH - 1):
        ps.append(((2 * l + 1) * zc * ps[-1] - l * ps[-2]) / (l + 1))
    cb = ps[0][None, :]
    for l in range(1, NUM_SPH):
        cb = jnp.where(lcol == l, ps[l][None, :], cb)

    full = res * n_ref[:] * env[None, :] * cb
    o_ref[:, sl] = full[:NCOL, :]


def kernel(dist, angle, idx_kj):
    E = dist.shape[0]
    T = angle.shape[0]

    sc_gather = _make_sc_gather(E, T)
    dist_t = sc_gather(dist, idx_kj)

    BT = 8192
    out = pl.pallas_call(
        _basis_body,
        grid=(pl.cdiv(T, BT),),
        in_specs=[
            pl.BlockSpec((NPAD, 1), lambda i: (0, 0)),
            pl.BlockSpec((NPAD, 1), lambda i: (0, 0)),
            pl.BlockSpec((BT,), lambda i: (i,)),
            pl.BlockSpec((BT,), lambda i: (i,)),
        ],
        out_specs=pl.BlockSpec((NCOL, BT), lambda i: (0, i)),
        out_shape=jax.ShapeDtypeStruct((NCOL, T), jnp.float32),
    )(jnp.asarray(_ZPAD)[:, None], jnp.asarray(_NORMPAD)[:, None], dist_t, angle)
    return out.T


# speedup vs baseline: 1.0895x; 1.0081x over previous
import functools

import jax
import jax.numpy as jnp
import numpy as np
from jax import lax
from jax.experimental import pallas as pl
from jax.experimental.pallas import tpu as pltpu
import jax.experimental.pallas.tpu_sc as plsc

NUM_SPH = 7
NUM_RAD = 6
CUTOFF = 5.0
P_ENV = 6
A_ENV = -(P_ENV + 1) * (P_ENV + 2) / 2.0
B_ENV = float(P_ENV * (P_ENV + 2))
C_ENV = -P_ENV * (P_ENV + 1) / 2.0

NCOL = NUM_SPH * NUM_RAD
NPAD = 48


def _jn_np(x, l):
    x = np.asarray(x, dtype=np.float64)
    j0 = np.sin(x) / x
    if l == 0:
        return j0
    j1 = np.sin(x) / x ** 2 - np.cos(x) / x
    if l == 1:
        return j1
    jm1, j = j0, j1
    for ll in range(1, l):
        jm1, j = j, (2 * ll + 1) / x * j - jm1
    return j


def _jn_zeros(n, k):
    zeros = np.zeros((n, k))
    for l in range(n):
        grid = np.linspace(0.5 + 0.5 * l, 45.0 + 5.0 * l, 200001)
        vals = _jn_np(grid, l)
        sign_change = np.where(np.sign(vals[:-1]) * np.sign(vals[1:]) < 0)[0][:k]
        for i, ii in enumerate(sign_change):
            lo, hi = grid[ii], grid[ii + 1]
            flo = _jn_np(lo, l)
            for _ in range(60):
                mid = 0.5 * (lo + hi)
                fmid = _jn_np(mid, l)
                if flo * fmid <= 0:
                    hi = mid
                else:
                    lo, flo = mid, fmid
            zeros[l, i] = 0.5 * (lo + hi)
    return zeros


_Z = _jn_zeros(NUM_SPH, NUM_RAD)
_NORM = np.stack([1.0 / np.sqrt(0.5 * _jn_np(_Z[l], l + 1) ** 2) for l in range(NUM_SPH)])
_SPH_PREF = np.array([np.sqrt((2 * l + 1) / (4 * np.pi)) for l in range(NUM_SPH)])

_ZPAD = np.ones((NPAD,), np.float32)
_ZPAD[:NCOL] = _Z.reshape(-1)
_NORMPAD = np.zeros((NPAD,), np.float32)
_NORMPAD[:NCOL] = _NORM.reshape(-1)
for _c in range(NCOL):
    _NORMPAD[_c] *= np.float32(_SPH_PREF[_c // NUM_RAD])


def _make_sc_gather(E, T):
    NC = 2
    NW = 32
    ROWS_W = T // NW
    SUB = 120
    CH = 3000
    NSUB = CH // SUB
    NCHUNK = ROWS_W // CH
    assert T == NW * NCHUNK * CH and CH == NSUB * SUB

    mesh = plsc.VectorSubcoreMesh(core_axis_name="c", subcore_axis_name="s")

    @functools.partial(
        pl.kernel,
        out_type=jax.ShapeDtypeStruct((T,), jnp.float32),
        mesh=mesh,
        scratch_types=[
            pltpu.VMEM((CH,), jnp.int32),
            pltpu.VMEM((CH,), jnp.float32),
            pltpu.SemaphoreType.DMA,
        ],
    )
    def sc_gather(dist_hbm, idx_hbm, out_hbm, idx_v, val_v, sem):
        wid = lax.axis_index("s") * NC + lax.axis_index("c")
        row0 = wid * ROWS_W

        def chunk_body(i, carry):
            base = row0 + i * CH
            pltpu.sync_copy(idx_hbm.at[pl.ds(base, CH)], idx_v)
            cps = [
                pltpu.async_copy(
                    dist_hbm.at[idx_v.at[pl.ds(j * SUB, SUB)]],
                    val_v.at[pl.ds(j * SUB, SUB)],
                    sem,
                )
                for j in range(NSUB)
            ]
            for cp in cps:
                cp.wait()
            pltpu.sync_copy(val_v, out_hbm.at[pl.ds(base, CH)])
            return carry

        lax.fori_loop(0, NCHUNK, chunk_body, 0)

    return sc_gather


_PIO2_HI = np.float32(1.57079625129699707031)
_PIO2_LO = np.float32(7.54978941586159635335e-08)
_S1, _S2, _S3 = np.float32(-1.6666654611e-1), np.float32(8.3321608736e-3), np.float32(-1.9515295891e-4)
_C1, _C2, _C3 = np.float32(-0.5), np.float32(4.166664568298827e-2), np.float32(-1.388731625493765e-3)
_C4 = np.float32(2.443315711809948e-5)


def _sincos(y):
    q = y * np.float32(2.0 / np.pi)
    nf = jnp.floor(q + 0.5)
    ni = nf.astype(jnp.int32)
    r = (y - nf * _PIO2_HI) - nf * _PIO2_LO
    r2 = r * r
    sp = r + r * r2 * (_S1 + r2 * (_S2 + r2 * _S3))
    cp = 1.0 + r2 * (_C1 + r2 * (_C2 + r2 * (_C3 + r2 * _C4)))
    odd = (ni & 1) == 1
    sin_sel = jnp.where(odd, cp, sp)
    cos_sel = jnp.where(odd, sp, cp)
    sbit = jax.lax.shift_left(ni & 2, 30)
    cbit = jax.lax.shift_left((ni + 1) & 2, 30)
    sin = lax.bitcast_convert_type(
        lax.bitcast_convert_type(sin_sel, jnp.int32) ^ sbit, jnp.float32
    )
    cos = lax.bitcast_convert_type(
        lax.bitcast_convert_type(cos_sel, jnp.int32) ^ cbit, jnp.float32
    )
    return sin, cos


def _basis_body(z_ref, n_ref, d_ref, a_ref, o_ref):
    for k in range(d_ref.shape[0] // 128):
        sl = pl.ds(k * 128, 128)
        _basis_chunk(z_ref, n_ref, d_ref[sl], a_ref[sl], o_ref, sl)


def _basis_chunk(z_ref, n_ref, d, a, o_ref, sl):
    x = d * (1.0 / CUTOFF)
    x2 = x * x
    x4 = x2 * x2
    x5 = x4 * x
    env = 1.0 / x + A_ENV * x5 + B_ENV * x5 * x + C_ENV * x5 * x2
    y = z_ref[:] * x[None, :]
    inv_y = 1.0 / y
    sy, cy = _sincos(y)
    j0 = sy * inv_y
    j1 = sy * inv_y * inv_y - cy * inv_y
    lcol = lax.broadcasted_iota(jnp.int32, y.shape, 0) // NUM_RAD
    res = jnp.where(lcol == 0, j0, j1)
    jm1, j = j0, j1
    for s in range(1, NUM_SPH - 1):
        jm1, j = j, (2 * s + 1) * inv_y * j - jm1
        res = jnp.where(lcol == s + 1, j, res)

    zc = _sincos(a)[1]
    ps = [jnp.ones_like(zc), zc]
    for l in range(1, NUM_SPH - 1):
        ps.append(((2 * l + 1) * zc * ps[-1] - l * ps[-2]) / (l + 1))
    cb = ps[0][None, :]
    for l in range(1, NUM_SPH):
        cb = jnp.where(lcol == l, ps[l][None, :], cb)

    full = res * n_ref[:] * env[None, :] * cb
    o_ref[:, sl] = full[:NCOL, :]


def kernel(dist, angle, idx_kj):
    E = dist.shape[0]
    T = angle.shape[0]

    sc_gather = _make_sc_gather(E, T)
    dist_t = sc_gather(dist, idx_kj)

    BT = 16384
    out = pl.pallas_call(
        _basis_body,
        grid=(pl.cdiv(T, BT),),
        in_specs=[
            pl.BlockSpec((NPAD, 1), lambda i: (0, 0)),
            pl.BlockSpec((NPAD, 1), lambda i: (0, 0)),
            pl.BlockSpec((BT,), lambda i: (i,)),
            pl.BlockSpec((BT,), lambda i: (i,)),
        ],
        out_specs=pl.BlockSpec((NCOL, BT), lambda i: (0, i)),
        out_shape=jax.ShapeDtypeStruct((NCOL, T), jnp.float32),
    )(jnp.asarray(_ZPAD)[:, None], jnp.asarray(_NORMPAD)[:, None], dist_t, angle)
    return out.T


# speedup vs baseline: 1.0989x; 1.0085x over previous
import functools

import jax
import jax.numpy as jnp
import numpy as np
from jax import lax
from jax.experimental import pallas as pl
from jax.experimental.pallas import tpu as pltpu
import jax.experimental.pallas.tpu_sc as plsc

NUM_SPH = 7
NUM_RAD = 6
CUTOFF = 5.0
P_ENV = 6
A_ENV = -(P_ENV + 1) * (P_ENV + 2) / 2.0
B_ENV = float(P_ENV * (P_ENV + 2))
C_ENV = -P_ENV * (P_ENV + 1) / 2.0

NCOL = NUM_SPH * NUM_RAD
NPAD = 48


def _jn_np(x, l):
    x = np.asarray(x, dtype=np.float64)
    j0 = np.sin(x) / x
    if l == 0:
        return j0
    j1 = np.sin(x) / x ** 2 - np.cos(x) / x
    if l == 1:
        return j1
    jm1, j = j0, j1
    for ll in range(1, l):
        jm1, j = j, (2 * ll + 1) / x * j - jm1
    return j


def _jn_zeros(n, k):
    zeros = np.zeros((n, k))
    for l in range(n):
        grid = np.linspace(0.5 + 0.5 * l, 45.0 + 5.0 * l, 200001)
        vals = _jn_np(grid, l)
        sign_change = np.where(np.sign(vals[:-1]) * np.sign(vals[1:]) < 0)[0][:k]
        for i, ii in enumerate(sign_change):
            lo, hi = grid[ii], grid[ii + 1]
            flo = _jn_np(lo, l)
            for _ in range(60):
                mid = 0.5 * (lo + hi)
                fmid = _jn_np(mid, l)
                if flo * fmid <= 0:
                    hi = mid
                else:
                    lo, flo = mid, fmid
            zeros[l, i] = 0.5 * (lo + hi)
    return zeros


_Z = _jn_zeros(NUM_SPH, NUM_RAD)
_NORM = np.stack([1.0 / np.sqrt(0.5 * _jn_np(_Z[l], l + 1) ** 2) for l in range(NUM_SPH)])
_SPH_PREF = np.array([np.sqrt((2 * l + 1) / (4 * np.pi)) for l in range(NUM_SPH)])

_ZPAD = np.ones((NPAD,), np.float32)
_ZPAD[:NCOL] = _Z.reshape(-1)
_NORMPAD = np.zeros((NPAD,), np.float32)
_NORMPAD[:NCOL] = _NORM.reshape(-1)
for _c in range(NCOL):
    _NORMPAD[_c] *= np.float32(_SPH_PREF[_c // NUM_RAD])


def _make_sc_gather(E, T):
    NC = 2
    NW = 32
    ROWS_W = T // NW
    SUB = 120
    CH = 3000
    NSUB = CH // SUB
    NCHUNK = ROWS_W // CH
    assert T == NW * NCHUNK * CH and CH == NSUB * SUB

    mesh = plsc.VectorSubcoreMesh(core_axis_name="c", subcore_axis_name="s")

    @functools.partial(
        pl.kernel,
        out_type=jax.ShapeDtypeStruct((T,), jnp.float32),
        mesh=mesh,
        scratch_types=[
            pltpu.VMEM((CH,), jnp.int32),
            pltpu.VMEM((CH,), jnp.float32),
            pltpu.SemaphoreType.DMA,
        ],
    )
    def sc_gather(dist_hbm, idx_hbm, out_hbm, idx_v, val_v, sem):
        wid = lax.axis_index("s") * NC + lax.axis_index("c")
        row0 = wid * ROWS_W

        def chunk_body(i, carry):
            base = row0 + i * CH
            pltpu.sync_copy(idx_hbm.at[pl.ds(base, CH)], idx_v)
            cps = [
                pltpu.async_copy(
                    dist_hbm.at[idx_v.at[pl.ds(j * SUB, SUB)]],
                    val_v.at[pl.ds(j * SUB, SUB)],
                    sem,
                )
                for j in range(NSUB)
            ]
            for cp in cps:
                cp.wait()
            pltpu.sync_copy(val_v, out_hbm.at[pl.ds(base, CH)])
            return carry

        lax.fori_loop(0, NCHUNK, chunk_body, 0)

    return sc_gather


_PIO2_HI = np.float32(1.57079625129699707031)
_PIO2_LO = np.float32(7.54978941586159635335e-08)
_S1, _S2, _S3 = np.float32(-1.6666654611e-1), np.float32(8.3321608736e-3), np.float32(-1.9515295891e-4)
_C1, _C2, _C3 = np.float32(-0.5), np.float32(4.166664568298827e-2), np.float32(-1.388731625493765e-3)
_C4 = np.float32(2.443315711809948e-5)


def _sincos(y):
    q = y * np.float32(2.0 / np.pi)
    nf = jnp.floor(q + 0.5)
    ni = nf.astype(jnp.int32)
    r = (y - nf * _PIO2_HI) - nf * _PIO2_LO
    r2 = r * r
    sp = r + r * r2 * (_S1 + r2 * (_S2 + r2 * _S3))
    cp = 1.0 + r2 * (_C1 + r2 * (_C2 + r2 * (_C3 + r2 * _C4)))
    odd = (ni & 1) == 1
    sin_sel = jnp.where(odd, cp, sp)
    cos_sel = jnp.where(odd, sp, cp)
    sbit = jax.lax.shift_left(ni & 2, 30)
    cbit = jax.lax.shift_left((ni + 1) & 2, 30)
    sin = lax.bitcast_convert_type(
        lax.bitcast_convert_type(sin_sel, jnp.int32) ^ sbit, jnp.float32
    )
    cos = lax.bitcast_convert_type(
        lax.bitcast_convert_type(cos_sel, jnp.int32) ^ cbit, jnp.float32
    )
    return sin, cos


def _basis_body(z_ref, n_ref, d_ref, a_ref, o_ref):
    for k in range(d_ref.shape[0] // 128):
        sl = pl.ds(k * 128, 128)
        _basis_chunk(z_ref, n_ref, d_ref[sl], a_ref[sl], o_ref, sl)


def _basis_chunk(z_ref, n_ref, d, a, o_ref, sl):
    x = d * (1.0 / CUTOFF)
    x2 = x * x
    x4 = x2 * x2
    x5 = x4 * x
    env = 1.0 / x + A_ENV * x5 + B_ENV * x5 * x + C_ENV * x5 * x2
    y = z_ref[:] * x[None, :]
    inv_y = 1.0 / y
    sy, cy = _sincos(y)
    j0 = sy * inv_y
    j1 = (j0 - cy) * inv_y
    lcol = lax.broadcasted_iota(jnp.int32, y.shape, 0) // NUM_RAD
    res = jnp.where(lcol == 0, j0, j1)
    jm1, j = j0, j1
    for s in range(1, NUM_SPH - 1):
        jm1, j = j, (2 * s + 1) * inv_y * j - jm1
        res = jnp.where(lcol == s + 1, j, res)

    zc = _sincos(a)[1]
    ps = [jnp.ones_like(zc), zc]
    for l in range(1, NUM_SPH - 1):
        ps.append(((2 * l + 1) * zc * ps[-1] - l * ps[-2]) / (l + 1))
    cb = ps[0][None, :]
    for l in range(1, NUM_SPH):
        cb = jnp.where(lcol == l, ps[l][None, :], cb)

    full = res * n_ref[:] * env[None, :] * cb
    o_ref[:, sl] = full[:NCOL, :]


def kernel(dist, angle, idx_kj):
    E = dist.shape[0]
    T = angle.shape[0]

    sc_gather = _make_sc_gather(E, T)
    dist_t = sc_gather(dist, idx_kj)

    BT = 16384
    out = pl.pallas_call(
        _basis_body,
        grid=(pl.cdiv(T, BT),),
        in_specs=[
            pl.BlockSpec((NPAD, 1), lambda i: (0, 0)),
            pl.BlockSpec((NPAD, 1), lambda i: (0, 0)),
            pl.BlockSpec((BT,), lambda i: (i,)),
            pl.BlockSpec((BT,), lambda i: (i,)),
        ],
        out_specs=pl.BlockSpec((NCOL, BT), lambda i: (0, i)),
        out_shape=jax.ShapeDtypeStruct((NCOL, T), jnp.float32),
    )(jnp.asarray(_ZPAD)[:, None], jnp.asarray(_NORMPAD)[:, None], dist_t, angle)
    return out.T
